# baseline (speedup 1.0000x reference)
import functools

import jax
import jax.numpy as jnp
from jax.experimental import pallas as pl

A = 5
N = 10000
E = 320000
D = 128
H = 8
HID = 64


def _segment_softmax(e, seg, num_segments):
    m = jax.ops.segment_max(e, seg, num_segments=num_segments)
    m = jnp.where(jnp.isfinite(m), m, 0.0)
    ex = jnp.exp(e - m[seg])
    s = jax.ops.segment_sum(ex, seg, num_segments=num_segments)
    return ex / (s[seg] + 1e-16)


def _gatconv(x, src, dst, W, a_src, a_dst, b, heads, out_dim, concat):
    n = x.shape[0]
    h = (x @ W).reshape(n, heads, out_dim)
    alpha_src = (h * a_src[None]).sum(-1)
    alpha_dst = (h * a_dst[None]).sum(-1)
    e = jax.nn.leaky_relu(alpha_src[src] + alpha_dst[dst], negative_slope=0.2)
    alpha = _segment_softmax(e, dst, n)
    msgs = alpha[:, :, None] * h[src]
    out = jax.ops.segment_sum(msgs, dst, num_segments=n)
    if concat:
        out = out.reshape(n, heads * out_dim)
    else:
        out = out.mean(axis=1)
    return out + b, alpha


def _final_body(proba_ref, atts_ref, logits_ref, att_ref):
    p = proba_ref[...]
    m = jnp.max(p, axis=1, keepdims=True)
    lse = jnp.log(jnp.sum(jnp.exp(p - m), axis=1, keepdims=True))
    logits = p - m - lse
    logits_ref[...] = logits
    best = jnp.argmax(logits[0])
    onehot = (jax.lax.broadcasted_iota(jnp.int32, (A, 1), 0) == best).astype(jnp.float32)
    att_ref[...] = jnp.sum(atts_ref[...] * onehot, axis=0, keepdims=True)


@jax.jit
def _finalize(proba, atts):
    logits2, att2 = pl.pallas_call(
        _final_body,
        out_shape=[
            jax.ShapeDtypeStruct((1, A), jnp.float32),
            jax.ShapeDtypeStruct((1, N), jnp.float32),
        ],
    )(proba.reshape(1, A), atts)
    return logits2.reshape(A), att2.reshape(N)


def kernel(x, edge_index, W1, att_src1, att_dst1, bias1, W2, att_src2, att_dst2, bias2):
    proba = []
    attns = []
    for i in range(A):
        src = edge_index[i, 0]
        dst = edge_index[i, 1]
        h1, _ = _gatconv(x[i], src, dst, W1, att_src1, att_dst1, bias1, H, HID, concat=True)
        h1 = jax.nn.relu(h1)
        h2, alpha2 = _gatconv(h1, src, dst, W2, att_src2, att_dst2, bias2, 1, 1, concat=False)
        proba.append(h2.mean())
        w = alpha2[:, 0]
        att = jnp.zeros((N,), jnp.float32).at[dst].add(w[dst])
        attns.append(att)
    proba_vec = jnp.stack(proba)
    atts = jnp.stack(attns)
    return _finalize(proba_vec, atts)


# speedup vs baseline: 19.2717x; 19.2717x over previous
import functools

import jax
import jax.numpy as jnp
from jax import lax
from jax.experimental import pallas as pl
from jax.experimental.pallas import tpu as pltpu
from jax.experimental.pallas import tpu_sc as plsc

A = 5
N = 10000
E = 320000
D = 128
H = 8
HID = 64

NW = 32
EPW = E // NW
K = 80
NCH = EPW // K
NPAD = 10240
RPT = NPAD // 16
NP = 10240

_MESH = plsc.VectorSubcoreMesh(core_axis_name="c", subcore_axis_name="s")
_SC_PARAMS = pltpu.CompilerParams(
    needs_layout_passes=False, use_tc_tiling_on_sc=False)


def _leaky(v):
    return jnp.where(v >= 0, v, 0.2 * v)


def _tc1_body(x_ref, w1_ref, asrc_ref, adst_ref, hc_ref, att_ref, cex_ref):
    c = pl.program_id(1)
    xb = x_ref[0]
    hc_ref[0, 0] = jnp.dot(xb, w1_ref[0],
                           preferred_element_type=jnp.float32)

    @pl.when(c == 0)
    def _():
        dn = (((1,), (1,)), ((), ()))
        at_s = lax.dot_general(asrc_ref[...], xb, dn,
                               preferred_element_type=jnp.float32)
        at_d = lax.dot_general(adst_ref[...], xb, dn,
                               preferred_element_type=jnp.float32)
        att_ref[0, 0] = at_s
        att_ref[0, 1] = at_d
        bound = _leaky(jnp.max(at_s, axis=1) + jnp.max(at_d, axis=1))
        cex_ref[...] = jnp.broadcast_to(bound[:, None], (8, 128))[None]


def _run_tc1(x, w1, asrc_m, adst_m):
    return pl.pallas_call(
        _tc1_body,
        grid=(A, 8),
        in_specs=[
            pl.BlockSpec((1, N, D), lambda a, c: (a, 0, 0)),
            pl.BlockSpec((1, D, HID), lambda a, c: (c, 0, 0)),
            pl.BlockSpec((8, D), lambda a, c: (0, 0)),
            pl.BlockSpec((8, D), lambda a, c: (0, 0)),
        ],
        out_specs=[
            pl.BlockSpec((1, 1, N, HID), lambda a, c: (a, c, 0, 0)),
            pl.BlockSpec((1, 2, 8, N), lambda a, c: (a, 0, 0, 0)),
            pl.BlockSpec((1, 8, 128), lambda a, c: (a, 0, 0)),
        ],
        out_shape=[jax.ShapeDtypeStruct((A, 8, N, HID), jnp.float32),
                   jax.ShapeDtypeStruct((A, 2, 8, N), jnp.float32),
                   jax.ShapeDtypeStruct((A, 8, 128), jnp.float32)],
    )(x, w1, asrc_m, adst_m)


@functools.partial(
    pl.kernel,
    out_type=[
        jax.ShapeDtypeStruct((A, 8, 2, NPAD, HID), jnp.float32),
        jax.ShapeDtypeStruct((A, 8, NW, NPAD), jnp.float32),
    ],
    mesh=_MESH,
    compiler_params=_SC_PARAMS,
    scratch_types=[
        pltpu.VMEM((N,), jnp.float32),
        pltpu.VMEM((N,), jnp.float32),
        pltpu.VMEM((NPAD,), jnp.float32),
        pltpu.VMEM((K,), jnp.int32),
        pltpu.VMEM((K,), jnp.int32),
        pltpu.VMEM((K,), jnp.int32),
        pltpu.VMEM((K, HID), jnp.float32),
        pltpu.VMEM((32, HID), jnp.float32),
        pltpu.VMEM((16,), jnp.float32),
        pltpu.VMEM_SHARED((NPAD, HID), jnp.float32),
    ],
)
def _conv1(hcf_h, att_h, src_h, dst_h, cexp_h, acc_out, s1_out,
           as_t, ad_t, s1a,
           srcb, srcb_off, dstb, rows, zb, cbuf, acc_sh):
    cax = lax.axis_index("c")
    sax = lax.axis_index("s")
    wid = sax * 2 + cax
    z16 = jnp.zeros((16,), jnp.float32)
    for r in range(32):
        for k in range(4):
            zb[r, pl.ds(k * 16, 16)] = z16

    def per_ac(a, c):
        pltpu.sync_copy(cexp_h.at[a, c], cbuf)
        pltpu.sync_copy(att_h.at[a, 0, c], as_t)
        pltpu.sync_copy(att_h.at[a, 1, c], ad_t)
        cv = cbuf[...]
        c0 = cv[0]
        roff = (a * 8 + c) * N

        def zero_s1(g, carry):
            s1a[pl.ds(g * 16, 16)] = z16
            return carry
        lax.fori_loop(0, NPAD // 16, zero_s1, None)

        def zero_acc(r, carry):
            pltpu.sync_copy(zb, acc_sh.at[pl.ds(sax * RPT + r * 32, 32)])
            return carry
        lax.fori_loop(0, RPT // 32, zero_acc, None)
        plsc.subcore_barrier()

        def chunk(t, carry):
            off = wid * EPW + t * K
            pltpu.sync_copy(src_h.at[a, pl.ds(off, K)], srcb)
            pltpu.sync_copy(dst_h.at[a, pl.ds(off, K)], dstb)

            def offs(g, cy):
                srcb_off[pl.ds(g * 16, 16)] = srcb[pl.ds(g * 16, 16)] + roff
                return cy
            lax.fori_loop(0, K // 16, offs, None)
            pltpu.sync_copy(hcf_h.at[srcb_off], rows)

            def group(g, cy):
                sv = srcb[pl.ds(g * 16, 16)]
                dv = dstb[pl.ds(g * 16, 16)]
                a_s = plsc.load_gather(as_t, [sv])
                a_d = plsc.load_gather(ad_t, [dv])
                ex0 = jnp.exp(_leaky(a_s + a_d) - c0)
                plsc.addupdate_scatter(s1a, [dv], ex0)
                for l in range(16):
                    j = g * 16 + l
                    s0 = ex0[l]
                    for k in range(4):
                        rows[j, pl.ds(k * 16, 16)] = (
                            rows[j, pl.ds(k * 16, 16)] * s0)
                return cy
            lax.fori_loop(0, K // 16, group, None)
            pltpu.sync_copy(rows, acc_sh.at[dstb], add=True)
            return carry
        lax.fori_loop(0, NCH, chunk, None)
        plsc.subcore_barrier()
        pltpu.sync_copy(s1a, s1_out.at[a, c, wid])
        pltpu.sync_copy(acc_sh.at[pl.ds(sax * RPT, RPT)],
                        acc_out.at[a, c, cax, pl.ds(sax * RPT, RPT)])
        plsc.subcore_barrier()

    lax.fori_loop(
        0, A,
        lambda a, _: lax.fori_loop(
            0, 8, lambda c, __: per_ac(a, c), None),
        None)


@functools.partial(
    pl.kernel,
    out_type=jax.ShapeDtypeStruct((A, 8, NPAD, HID), jnp.float32),
    mesh=_MESH,
    compiler_params=_SC_PARAMS,
    scratch_types=[
        pltpu.VMEM((160, HID), jnp.float32),
        pltpu.VMEM((160, HID), jnp.float32),
        pltpu.VMEM((NW, 320), jnp.float32),
        pltpu.VMEM((320,), jnp.float32),
    ],
)
def _norm1(accr_h, s1_h, accn_out, abuf, bbuf, sbuf, stot):
    cax = lax.axis_index("c")
    sax = lax.axis_index("s")
    wid = sax * 2 + cax
    base = wid * 320

    def per_ac(a, c):
        pltpu.sync_copy(s1_h.at[a, c, :, pl.ds(base, 320)], sbuf)

        def sred(g, carry):
            tot = jnp.zeros((16,), jnp.float32)
            for w in range(NW):
                tot = tot + sbuf[w, pl.ds(g * 16, 16)]
            stot[pl.ds(g * 16, 16)] = 1.0 / (tot + 1e-16)
            return carry
        lax.fori_loop(0, 20, sred, None)

        def ndiv(q, carry):
            pltpu.sync_copy(accr_h.at[a, c, 0, pl.ds(base + q * 160, 160)],
                            abuf)
            pltpu.sync_copy(accr_h.at[a, c, 1, pl.ds(base + q * 160, 160)],
                            bbuf)

            def ngrp(g, cy):
                rv = stot[pl.ds(q * 160 + g * 16, 16)]
                for l in range(16):
                    j = g * 16 + l
                    r = rv[l]
                    for k in range(4):
                        abuf[j, pl.ds(k * 16, 16)] = (
                            (abuf[j, pl.ds(k * 16, 16)]
                             + bbuf[j, pl.ds(k * 16, 16)]) * r)
                return cy
            lax.fori_loop(0, 10, ngrp, None)
            pltpu.sync_copy(
                abuf, accn_out.at[a, c, pl.ds(base + q * 160, 160)])
            return carry
        lax.fori_loop(0, 2, ndiv, None)

    lax.fori_loop(
        0, A,
        lambda a, _: lax.fori_loop(
            0, 8, lambda c, __: per_ac(a, c), None),
        None)


def _tc2_body(acc_ref, b1_ref, w2_ref, as2_ref, ad2_ref,
              h2_ref, c2_ref):
    c = pl.program_id(1)
    h1c = jnp.maximum(acc_ref[0, 0] + b1_ref[0], 0.0)
    dn = (((0,), (1,)), ((), ()))
    part = lax.dot_general(w2_ref[...], h1c, dn,
                           preferred_element_type=jnp.float32)

    @pl.when(c == 0)
    def _():
        h2_ref[...] = part[None]

    @pl.when(c > 0)
    def _():
        h2_ref[...] = h2_ref[...] + part[None]

    @pl.when(c == 7)
    def _():
        h2 = h2_ref[0]
        as2 = as2_ref[0, 0]
        ad2 = ad2_ref[0, 0]
        hmx = jnp.max(h2)
        hmn = jnp.min(h2)
        bound = (jnp.maximum(as2 * hmx, as2 * hmn)
                 + jnp.maximum(ad2 * hmx, ad2 * hmn))
        c2_ref[...] = jnp.broadcast_to(_leaky(bound), (1, 1, 128))


def _run_tc2(acc, b1, w2, as2p, ad2p):
    return pl.pallas_call(
        _tc2_body,
        grid=(A, 8),
        in_specs=[
            pl.BlockSpec((1, 1, NPAD, HID), lambda a, c: (a, c, 0, 0)),
            pl.BlockSpec((1, 1, HID), lambda a, c: (c, 0, 0)),
            pl.BlockSpec((HID, 1), lambda a, c: (c, 0)),
            pl.BlockSpec((1, 128), lambda a, c: (0, 0)),
            pl.BlockSpec((1, 128), lambda a, c: (0, 0)),
        ],
        out_specs=[
            pl.BlockSpec((1, 1, NPAD), lambda a, c: (a, 0, 0)),
            pl.BlockSpec((1, 1, 128), lambda a, c: (a, 0, 0)),
        ],
        out_shape=[jax.ShapeDtypeStruct((A, 1, NPAD), jnp.float32),
                   jax.ShapeDtypeStruct((A, 1, 128), jnp.float32)],
    )(acc, b1, w2, as2p, ad2p)


@functools.partial(
    pl.kernel,
    out_type=[
        jax.ShapeDtypeStruct((A, NW, 3, N), jnp.float32),
        jax.ShapeDtypeStruct((A, N), jnp.float32),
    ],
    mesh=_MESH,
    compiler_params=_SC_PARAMS,
    scratch_types=[
        pltpu.VMEM((N,), jnp.float32),
        pltpu.VMEM((EPW,), jnp.int32),
        pltpu.VMEM((EPW,), jnp.int32),
        pltpu.VMEM((N,), jnp.float32),
        pltpu.VMEM((N,), jnp.float32),
        pltpu.VMEM((N,), jnp.float32),
        pltpu.VMEM((EPW,), jnp.float32),
        pltpu.VMEM((16,), jnp.float32),
    ],
)
def _conv2(h2f_h, src_h, dst_h, prm_h, t2_out, exf_out,
           h2v, srcb, dstb, s2v, numv, indv, exb, pbuf):
    cax = lax.axis_index("c")
    sax = lax.axis_index("s")
    wid = sax * 2 + cax
    z16 = jnp.zeros((16,), jnp.float32)
    one16 = jnp.ones((16,), jnp.float32)
    for a in range(A):
        pltpu.sync_copy(h2f_h.at[a], h2v)
        pltpu.sync_copy(src_h.at[a, pl.ds(wid * EPW, EPW)], srcb)
        pltpu.sync_copy(dst_h.at[a, pl.ds(wid * EPW, EPW)], dstb)
        pltpu.sync_copy(prm_h.at[a], pbuf)
        pv = pbuf[...]
        as2 = pv[0]
        ad2 = pv[1]
        c2 = pv[2]

        def zero(g, carry):
            s2v[pl.ds(g * 16, 16)] = z16
            numv[pl.ds(g * 16, 16)] = z16
            indv[pl.ds(g * 16, 16)] = z16
            return carry
        lax.fori_loop(0, N // 16, zero, None)

        def group(g, carry):
            sv = srcb[pl.ds(g * 16, 16)]
            dv = dstb[pl.ds(g * 16, 16)]
            hs = plsc.load_gather(h2v, [sv])
            hd = plsc.load_gather(h2v, [dv])
            exv = jnp.exp(_leaky(hs * as2 + hd * ad2) - c2)
            plsc.addupdate_scatter(s2v, [dv], exv)
            plsc.addupdate_scatter(numv, [dv], exv * hs)
            plsc.addupdate_scatter(indv, [dv], one16)
            exb[pl.ds(g * 16, 16)] = exv
            return carry
        lax.fori_loop(0, EPW // 16, group, None)
        pltpu.sync_copy(s2v, t2_out.at[a, wid, 0])
        pltpu.sync_copy(numv, t2_out.at[a, wid, 1])
        pltpu.sync_copy(indv, t2_out.at[a, wid, 2])

        @pl.when(wid == 0)
        def _():
            pltpu.sync_copy(exb, exf_out.at[a])


def _tc3_body(t2_ref, exf_ref, b2_ref, s2s_ref, ie_ref, pacc_ref):
    sums = jnp.sum(t2_ref[0], axis=0)
    s2blk = sums[0:1]
    numblk = sums[1:2]
    indblk = sums[2:3]
    b2 = b2_ref[0, 0]
    o2 = numblk / (s2blk + 1e-16) + b2
    s2s_ref[...] = s2blk[None]
    ie_ref[...] = (indblk * exf_ref[0])[None]
    pacc_ref[...] = jnp.broadcast_to(jnp.sum(o2), (1, 1, 128))


def _run_tc3(t2p, exf, b2p):
    return pl.pallas_call(
        _tc3_body,
        grid=(A,),
        in_specs=[
            pl.BlockSpec((1, NW, 3, N), lambda a: (a, 0, 0, 0)),
            pl.BlockSpec((1, 1, N), lambda a: (a, 0, 0)),
            pl.BlockSpec((1, 128), lambda a: (0, 0)),
        ],
        out_specs=[
            pl.BlockSpec((1, 1, N), lambda a: (a, 0, 0)),
            pl.BlockSpec((1, 1, N), lambda a: (a, 0, 0)),
            pl.BlockSpec((1, 1, 128), lambda a: (a, 0, 0)),
        ],
        out_shape=[jax.ShapeDtypeStruct((A, 1, N), jnp.float32),
                   jax.ShapeDtypeStruct((A, 1, N), jnp.float32),
                   jax.ShapeDtypeStruct((A, 1, 128), jnp.float32)],
    )(t2p, exf, b2p)


@functools.partial(
    pl.kernel,
    out_type=jax.ShapeDtypeStruct((A, NP), jnp.float32),
    mesh=_MESH,
    compiler_params=_SC_PARAMS,
    scratch_types=[
        pltpu.VMEM((N,), jnp.float32),
        pltpu.VMEM((320,), jnp.float32),
        pltpu.VMEM((320,), jnp.int32),
        pltpu.VMEM((320,), jnp.float32),
    ],
)
def _att_kernel(s2s_h, iep_h, dstp_h, att_out, s2v, iev, dv, ob):
    cax = lax.axis_index("c")
    sax = lax.axis_index("s")
    wid = sax * 2 + cax
    base = wid * (NP // NW)
    for a in range(A):
        pltpu.sync_copy(s2s_h.at[a], s2v)
        pltpu.sync_copy(iep_h.at[a, pl.ds(base, NP // NW)], iev)
        pltpu.sync_copy(dstp_h.at[a, pl.ds(base, NP // NW)], dv)

        def group(g, carry):
            d16 = dv[pl.ds(g * 16, 16)]
            sg = plsc.load_gather(s2v, [d16])
            ob[pl.ds(g * 16, 16)] = iev[pl.ds(g * 16, 16)] / (sg + 1e-16)
            return carry
        lax.fori_loop(0, (NP // NW) // 16, group, None)
        pltpu.sync_copy(ob, att_out.at[a, pl.ds(base, NP // NW)])


def _final_body(proba_ref, atts_ref, logits_ref, att_ref):
    p = proba_ref[...] * (1.0 / N)
    m = jnp.max(p, axis=1, keepdims=True)
    lse = jnp.log(jnp.sum(jnp.exp(p - m), axis=1, keepdims=True))
    logits = p - m - lse
    logits_ref[...] = logits
    best = jnp.argmax(logits[0])
    onehot = (jax.lax.broadcasted_iota(jnp.int32, (A, 1), 0) == best)
    att_ref[...] = jnp.sum(atts_ref[...] * onehot.astype(jnp.float32),
                           axis=0, keepdims=True)


def _finalize(proba, atts):
    logits2, att2 = pl.pallas_call(
        _final_body,
        out_shape=[
            jax.ShapeDtypeStruct((1, A), jnp.float32),
            jax.ShapeDtypeStruct((1, N), jnp.float32),
        ],
    )(proba.reshape(1, A), atts)
    return logits2.reshape(A), att2.reshape(N)


def kernel(x, edge_index, W1, att_src1, att_dst1, bias1,
           W2, att_src2, att_dst2, bias2):
    ei = edge_index.astype(jnp.int32)
    src = ei[:, 0, :]
    dst = ei[:, 1, :]
    w1r = W1.reshape(D, H, HID)
    asrc_m = jnp.einsum("dhk,hk->hd", w1r, att_src1)
    adst_m = jnp.einsum("dhk,hk->hd", w1r, att_dst1)
    w1c = jnp.transpose(w1r, (1, 0, 2))

    hc, at_t, c1ex = _run_tc1(x, w1c, asrc_m, adst_m)
    cvals = c1ex[:, :, 0].reshape(A, 8, 1)
    cexp = jnp.pad(cvals, ((0, 0), (0, 0), (0, 15)))

    hcf = hc.reshape(A * 8 * N, HID)
    accr, s1p = _conv1(hcf, at_t, src, dst, cexp)
    acc = _norm1(accr, s1p)

    b1r = bias1.reshape(H, 1, HID)
    as2p = jnp.broadcast_to(att_src2.reshape(1, 1), (1, 128))
    ad2p = jnp.broadcast_to(att_dst2.reshape(1, 1), (1, 128))
    h2f, c2o = _run_tc2(acc, b1r, W2, as2p, ad2p)

    prm = jnp.concatenate(
        [jnp.broadcast_to(att_src2.reshape(1, 1), (A, 1)),
         jnp.broadcast_to(att_dst2.reshape(1, 1), (A, 1)),
         c2o[:, 0, 0:1],
         jnp.zeros((A, 13), jnp.float32)], axis=1)
    t2p, exf = _conv2(h2f.reshape(A, NPAD)[:, :N], src, dst, prm)

    b2p = jnp.broadcast_to(bias2.reshape(1, 1), (1, 128))
    s2s, ie, pacc = _run_tc3(t2p, exf.reshape(A, 1, N), b2p)

    iep = jnp.pad(ie.reshape(A, N), ((0, 0), (0, NP - N)))
    dstp = jnp.pad(dst[:, :N], ((0, 0), (0, NP - N)))
    attp = _att_kernel(s2s.reshape(A, N), iep, dstp)

    proba_raw = pacc[:, 0, 0]
    atts = attp[:, :N]
    return _finalize(proba_raw, atts)
